# baseline (speedup 1.0000x reference)
import functools

import jax
import jax.numpy as jnp
from jax.experimental import pallas as pl
from jax.experimental.pallas import tpu as pltpu

_TOPK = 2048
_SCALE = 0.08838834764831845
_NEG = -1e30
_FILL = -1

_B, _H, _D, _KV = 64, 32, 128, 4096
_BS = 128
_BPS = _KV // _BS


def _scores_body(kvlen_ref, qT_ref, k_ref, w_ref, kc_ref, ks_ref, out_ref):
    b = pl.program_id(0)
    seqlen = jnp.maximum(kvlen_ref[b], 1)
    p = seqlen - 1
    kd = kc_ref[...] * ks_ref[...][:, :, None]
    K2 = kd.reshape(_KV, _D)
    qh = qT_ref[0]
    ltT = jax.lax.dot_general(K2, qh, (((1,), (1,)), ((), ())),
                              preferred_element_type=jnp.float32)
    r = jnp.maximum(ltT * _SCALE, 0.0).astype(jnp.bfloat16)
    w = w_ref[0].astype(jnp.bfloat16)
    scores = jax.lax.dot_general(w, r, (((1,), (1,)), ((), ())),
                                 preferred_element_type=jnp.float32)
    kb = k_ref[0]
    qk = jax.lax.dot_general(kb, qh, (((1,), (1,)), ((), ())),
                             preferred_element_type=jnp.float32)
    rc = jnp.maximum(qk * _SCALE, 0.0).astype(jnp.bfloat16)
    cur = jax.lax.dot_general(rc, w, (((1,), (1,)), ((), ())),
                              preferred_element_type=jnp.float32)
    col = jax.lax.broadcasted_iota(jnp.int32, (1, _KV), 1)
    scores = jnp.where(col == p, cur, scores)
    out_ref[...] = jnp.where(col < seqlen, scores, _NEG).reshape(1, 1, _KV)


_R = 8
_HALF = 2048


def _halfsort_body(s_ref, vals_ref, idx_ref, *, off, asc):
    keys = s_ref[...]
    colv = jax.lax.broadcasted_iota(jnp.int32, (_R, _HALF), 1)
    idx = colv + off
    k = 2
    while k <= _HALF:
        j = k // 2
        while j >= 1:
            is_lower = (colv & j) == 0
            keep_max = (((colv & k) == 0) == is_lower) ^ asc
            pk = jnp.where(is_lower, jnp.roll(keys, -j, axis=1),
                           jnp.roll(keys, j, axis=1))
            pi = jnp.where(is_lower, jnp.roll(idx, -j, axis=1),
                           jnp.roll(idx, j, axis=1))
            beats = (pk > keys) | ((pk == keys) & (pi < idx))
            sel = ~(beats ^ keep_max)
            keys = jnp.where(sel, pk, keys)
            idx = jnp.where(sel, pi, idx)
            j //= 2
        k *= 2
    vals_ref[...] = keys
    idx_ref[...] = idx


def _merge_body(kv_ref, va_ref, ia_ref, vb_ref, ib_ref, vals_ref, idx_ref):
    A = va_ref[...]
    Ai = ia_ref[...]
    B = vb_ref[...]
    Bi = ib_ref[...]
    bwins = (B > A) | ((B == A) & (Bi < Ai))
    keys = jnp.where(bwins, B, A)
    idx = jnp.where(bwins, Bi, Ai)
    colv = jax.lax.broadcasted_iota(jnp.int32, (_R, _HALF), 1)
    j = _HALF // 2
    while j >= 1:
        is_lower = (colv & j) == 0
        pk = jnp.where(is_lower, jnp.roll(keys, -j, axis=1),
                       jnp.roll(keys, j, axis=1))
        pi = jnp.where(is_lower, jnp.roll(idx, -j, axis=1),
                       jnp.roll(idx, j, axis=1))
        beats = (pk > keys) | ((pk == keys) & (pi < idx))
        sel = ~(beats ^ is_lower)
        keys = jnp.where(sel, pk, keys)
        idx = jnp.where(sel, pi, idx)
        j //= 2
    kv = jnp.maximum(kv_ref[...], 1)
    ranks = jax.lax.broadcasted_iota(jnp.int32, (_R, _TOPK), 1)
    vals_ref[...] = keys
    idx_ref[...] = jnp.where(ranks < jnp.minimum(kv, _TOPK), idx, _FILL)


def kernel(q, k, weights, k_cache, k_s_cache, block_offsets, kv_seqlens):
    B, H, D = q.shape
    KV = block_offsets.shape[1] * k_cache.shape[1]
    kvl = kv_seqlens.astype(jnp.int32)

    scores = pl.pallas_call(
        _scores_body,
        grid_spec=pltpu.PrefetchScalarGridSpec(
            num_scalar_prefetch=1,
            grid=(B,),
            in_specs=[
                pl.BlockSpec((1, H, D), lambda b, kv: (b, 0, 0)),
                pl.BlockSpec((1, 1, D), lambda b, kv: (b, 0, 0)),
                pl.BlockSpec((1, 1, H), lambda b, kv: (b, 0, 0)),
                pl.BlockSpec((_BPS, _BS, D), lambda b, kv: (b, 0, 0)),
                pl.BlockSpec((_BPS, _BS), lambda b, kv: (b, 0)),
            ],
            out_specs=pl.BlockSpec((1, 1, KV), lambda b, kv: (b, 0, 0)),
        ),
        out_shape=jax.ShapeDtypeStruct((B, 1, KV), jnp.float32),
    )(kvl, q, k.reshape(B, 1, D), weights.reshape(B, 1, H),
      k_cache, k_s_cache)
    scores = scores.reshape(B, KV)

    halves = []
    for h, asc in ((0, False), (1, True)):
        hv, hi = pl.pallas_call(
            functools.partial(_halfsort_body, off=h * _HALF, asc=asc),
            grid=(B // _R,),
            in_specs=[
                pl.BlockSpec((_R, _HALF), lambda i, h=h: (i, h)),
            ],
            out_specs=[
                pl.BlockSpec((_R, _HALF), lambda i: (i, 0)),
                pl.BlockSpec((_R, _HALF), lambda i: (i, 0)),
            ],
            out_shape=[
                jax.ShapeDtypeStruct((B, _HALF), jnp.float32),
                jax.ShapeDtypeStruct((B, _HALF), jnp.int32),
            ],
        )(scores)
        halves.extend((hv, hi))

    vals, idx = pl.pallas_call(
        _merge_body,
        grid=(B // _R,),
        in_specs=[
            pl.BlockSpec((_R, 1), lambda i: (i, 0)),
            pl.BlockSpec((_R, _HALF), lambda i: (i, 0)),
            pl.BlockSpec((_R, _HALF), lambda i: (i, 0)),
            pl.BlockSpec((_R, _HALF), lambda i: (i, 0)),
            pl.BlockSpec((_R, _HALF), lambda i: (i, 0)),
        ],
        out_specs=[
            pl.BlockSpec((_R, _TOPK), lambda i: (i, 0)),
            pl.BlockSpec((_R, _TOPK), lambda i: (i, 0)),
        ],
        out_shape=[
            jax.ShapeDtypeStruct((B, _TOPK), jnp.float32),
            jax.ShapeDtypeStruct((B, _TOPK), jnp.int32),
        ],
    )(kvl.reshape(B, 1), *halves)

    return idx, vals


# speedup vs baseline: 1.0774x; 1.0774x over previous
import functools

import jax
import jax.numpy as jnp
from jax.experimental import pallas as pl
from jax.experimental.pallas import tpu as pltpu

_TOPK = 2048
_SCALE = 0.08838834764831845
_NEG = -1e30
_FILL = -1

_B, _H, _D, _KV = 64, 32, 128, 4096
_BS = 128
_BPS = _KV // _BS


_SEQ = 2


def _scores_body(kvlen_ref, qT_ref, k_ref, w_ref, kc_ref, ks_ref, out_ref):
    b = pl.program_id(0)
    col = jax.lax.broadcasted_iota(jnp.int32, (1, _KV), 1)
    for s in range(_SEQ):
        seqlen = jnp.maximum(kvlen_ref[b * _SEQ + s], 1)
        p = seqlen - 1
        kd = (kc_ref[s * _BPS:(s + 1) * _BPS]
              * ks_ref[s * _BPS:(s + 1) * _BPS][:, :, None])
        K2 = kd.reshape(_KV, _D)
        qh = qT_ref[s]
        ltT = jax.lax.dot_general(K2, qh, (((1,), (1,)), ((), ())),
                                  preferred_element_type=jnp.float32)
        r = jnp.maximum(ltT * _SCALE, 0.0).astype(jnp.bfloat16)
        w = w_ref[s].astype(jnp.bfloat16)
        scores = jax.lax.dot_general(w, r, (((1,), (1,)), ((), ())),
                                     preferred_element_type=jnp.float32)
        kb = k_ref[s]
        qk = jax.lax.dot_general(kb, qh, (((1,), (1,)), ((), ())),
                                 preferred_element_type=jnp.float32)
        rc = jnp.maximum(qk * _SCALE, 0.0).astype(jnp.bfloat16)
        cur = jax.lax.dot_general(rc, w, (((1,), (1,)), ((), ())),
                                  preferred_element_type=jnp.float32)
        scores = jnp.where(col == p, cur, scores)
        out_ref[s] = jnp.where(col < seqlen, scores, _NEG)


_R = 8
_HALF = 2048


def _halfsort_body(s_ref, vals_ref, idx_ref, *, off, asc):
    keys = s_ref[...]
    colv = jax.lax.broadcasted_iota(jnp.int32, (_R, _HALF), 1)
    idx = colv + off
    k = 2
    while k <= _HALF:
        j = k // 2
        while j >= 1:
            is_lower = (colv & j) == 0
            keep_max = (((colv & k) == 0) == is_lower) ^ asc
            pk = jnp.where(is_lower, jnp.roll(keys, -j, axis=1),
                           jnp.roll(keys, j, axis=1))
            pi = jnp.where(is_lower, jnp.roll(idx, -j, axis=1),
                           jnp.roll(idx, j, axis=1))
            beats = (pk > keys) | ((pk == keys) & (pi < idx))
            sel = ~(beats ^ keep_max)
            keys = jnp.where(sel, pk, keys)
            idx = jnp.where(sel, pi, idx)
            j //= 2
        k *= 2
    vals_ref[...] = keys
    idx_ref[...] = idx


def _merge_body(kv_ref, va_ref, ia_ref, vb_ref, ib_ref, vals_ref, idx_ref):
    A = va_ref[...]
    Ai = ia_ref[...]
    B = vb_ref[...]
    Bi = ib_ref[...]
    bwins = (B > A) | ((B == A) & (Bi < Ai))
    keys = jnp.where(bwins, B, A)
    idx = jnp.where(bwins, Bi, Ai)
    colv = jax.lax.broadcasted_iota(jnp.int32, (_R, _HALF), 1)
    j = _HALF // 2
    while j >= 1:
        is_lower = (colv & j) == 0
        pk = jnp.where(is_lower, jnp.roll(keys, -j, axis=1),
                       jnp.roll(keys, j, axis=1))
        pi = jnp.where(is_lower, jnp.roll(idx, -j, axis=1),
                       jnp.roll(idx, j, axis=1))
        beats = (pk > keys) | ((pk == keys) & (pi < idx))
        sel = ~(beats ^ is_lower)
        keys = jnp.where(sel, pk, keys)
        idx = jnp.where(sel, pi, idx)
        j //= 2
    kv = jnp.maximum(kv_ref[...], 1)
    ranks = jax.lax.broadcasted_iota(jnp.int32, (_R, _TOPK), 1)
    vals_ref[...] = keys
    idx_ref[...] = jnp.where(ranks < jnp.minimum(kv, _TOPK), idx, _FILL)


def kernel(q, k, weights, k_cache, k_s_cache, block_offsets, kv_seqlens):
    B, H, D = q.shape
    KV = block_offsets.shape[1] * k_cache.shape[1]
    kvl = kv_seqlens.astype(jnp.int32)

    scores = pl.pallas_call(
        _scores_body,
        grid_spec=pltpu.PrefetchScalarGridSpec(
            num_scalar_prefetch=1,
            grid=(B // _SEQ,),
            in_specs=[
                pl.BlockSpec((_SEQ, H, D), lambda b, kv: (b, 0, 0)),
                pl.BlockSpec((_SEQ, 1, D), lambda b, kv: (b, 0, 0)),
                pl.BlockSpec((_SEQ, 1, H), lambda b, kv: (b, 0, 0)),
                pl.BlockSpec((_SEQ * _BPS, _BS, D), lambda b, kv: (b, 0, 0)),
                pl.BlockSpec((_SEQ * _BPS, _BS), lambda b, kv: (b, 0)),
            ],
            out_specs=pl.BlockSpec((_SEQ, 1, KV), lambda b, kv: (b, 0, 0)),
        ),
        out_shape=jax.ShapeDtypeStruct((B, 1, KV), jnp.float32),
    )(kvl, q, k.reshape(B, 1, D), weights.reshape(B, 1, H),
      k_cache, k_s_cache)
    scores = scores.reshape(B, KV)

    halves = []
    for h, asc in ((0, False), (1, True)):
        hv, hi = pl.pallas_call(
            functools.partial(_halfsort_body, off=h * _HALF, asc=asc),
            grid=(B // _R,),
            in_specs=[
                pl.BlockSpec((_R, _HALF), lambda i, h=h: (i, h)),
            ],
            out_specs=[
                pl.BlockSpec((_R, _HALF), lambda i: (i, 0)),
                pl.BlockSpec((_R, _HALF), lambda i: (i, 0)),
            ],
            out_shape=[
                jax.ShapeDtypeStruct((B, _HALF), jnp.float32),
                jax.ShapeDtypeStruct((B, _HALF), jnp.int32),
            ],
        )(scores)
        halves.extend((hv, hi))

    vals, idx = pl.pallas_call(
        _merge_body,
        grid=(B // _R,),
        in_specs=[
            pl.BlockSpec((_R, 1), lambda i: (i, 0)),
            pl.BlockSpec((_R, _HALF), lambda i: (i, 0)),
            pl.BlockSpec((_R, _HALF), lambda i: (i, 0)),
            pl.BlockSpec((_R, _HALF), lambda i: (i, 0)),
            pl.BlockSpec((_R, _HALF), lambda i: (i, 0)),
        ],
        out_specs=[
            pl.BlockSpec((_R, _TOPK), lambda i: (i, 0)),
            pl.BlockSpec((_R, _TOPK), lambda i: (i, 0)),
        ],
        out_shape=[
            jax.ShapeDtypeStruct((B, _TOPK), jnp.float32),
            jax.ShapeDtypeStruct((B, _TOPK), jnp.int32),
        ],
    )(kvl.reshape(B, 1), *halves)

    return idx, vals


# speedup vs baseline: 1.4162x; 1.3144x over previous
import functools

import jax
import jax.numpy as jnp
from jax.experimental import pallas as pl
from jax.experimental.pallas import tpu as pltpu

_TOPK = 2048
_SCALE = 0.08838834764831845
_NEG = -1e30
_FILL = -1

_B, _H, _D, _KV = 64, 32, 128, 4096
_BS = 128
_BPS = _KV // _BS


_SEQ = 2


def _scores_body(kvlen_ref, qT_ref, k_ref, w_ref, kc_ref, ks_ref, out_ref):
    b = pl.program_id(0)
    col = jax.lax.broadcasted_iota(jnp.int32, (1, _KV), 1)
    for s in range(_SEQ):
        seqlen = jnp.maximum(kvlen_ref[b * _SEQ + s], 1)
        p = seqlen - 1
        kd = (kc_ref[s * _BPS:(s + 1) * _BPS]
              * ks_ref[s * _BPS:(s + 1) * _BPS][:, :, None])
        K2 = kd.reshape(_KV, _D)
        qh = qT_ref[s]
        ltT = jax.lax.dot_general(K2, qh, (((1,), (1,)), ((), ())),
                                  preferred_element_type=jnp.float32)
        r = jnp.maximum(ltT * _SCALE, 0.0).astype(jnp.bfloat16)
        w = w_ref[s].astype(jnp.bfloat16)
        scores = jax.lax.dot_general(w, r, (((1,), (1,)), ((), ())),
                                     preferred_element_type=jnp.float32)
        kb = k_ref[s]
        qk = jax.lax.dot_general(kb, qh, (((1,), (1,)), ((), ())),
                                 preferred_element_type=jnp.float32)
        rc = jnp.maximum(qk * _SCALE, 0.0).astype(jnp.bfloat16)
        cur = jax.lax.dot_general(rc, w, (((1,), (1,)), ((), ())),
                                  preferred_element_type=jnp.float32)
        scores = jnp.where(col == p, cur, scores)
        out_ref[s] = jnp.where(col < seqlen, scores, _NEG)


_R = 8
_HALF = 2048


def _cmpx(keys, idx, j, is_lower, keep_max):
    pk = jnp.where(is_lower, jnp.roll(keys, -j, axis=1),
                   jnp.roll(keys, j, axis=1))
    pi = jnp.where(is_lower, jnp.roll(idx, -j, axis=1),
                   jnp.roll(idx, j, axis=1))
    beats = (pk > keys) | ((pk == keys) & (pi < idx))
    sel = ~(beats ^ keep_max)
    return jnp.where(sel, pk, keys), jnp.where(sel, pi, idx)


def _topk_body(kv_ref, s_ref, vals_ref, idx_ref):
    s = s_ref[...]
    colv = jax.lax.broadcasted_iota(jnp.int32, (_R, _HALF), 1)
    ka = s[:, :_HALF]
    kb = s[:, _HALF:]
    ia = colv
    ib = colv + _HALF
    k = 2
    while k <= _HALF:
        j = k // 2
        while j >= 1:
            is_lower = (colv & j) == 0
            km = ((colv & k) == 0) == is_lower
            ka, ia = _cmpx(ka, ia, j, is_lower, km)
            kb, ib = _cmpx(kb, ib, j, is_lower, ~km)
            j //= 2
        k *= 2
    bwins = (kb > ka) | ((kb == ka) & (ib < ia))
    keys = jnp.where(bwins, kb, ka)
    idx = jnp.where(bwins, ib, ia)
    j = _HALF // 2
    while j >= 1:
        is_lower = (colv & j) == 0
        keys, idx = _cmpx(keys, idx, j, is_lower, is_lower)
        j //= 2
    kv = jnp.maximum(kv_ref[...], 1)
    ranks = jax.lax.broadcasted_iota(jnp.int32, (_R, _TOPK), 1)
    vals_ref[...] = keys
    idx_ref[...] = jnp.where(ranks < jnp.minimum(kv, _TOPK), idx, _FILL)


def kernel(q, k, weights, k_cache, k_s_cache, block_offsets, kv_seqlens):
    B, H, D = q.shape
    KV = block_offsets.shape[1] * k_cache.shape[1]
    kvl = kv_seqlens.astype(jnp.int32)

    scores = pl.pallas_call(
        _scores_body,
        grid_spec=pltpu.PrefetchScalarGridSpec(
            num_scalar_prefetch=1,
            grid=(B // _SEQ,),
            in_specs=[
                pl.BlockSpec((_SEQ, H, D), lambda b, kv: (b, 0, 0)),
                pl.BlockSpec((_SEQ, 1, D), lambda b, kv: (b, 0, 0)),
                pl.BlockSpec((_SEQ, 1, H), lambda b, kv: (b, 0, 0)),
                pl.BlockSpec((_SEQ * _BPS, _BS, D), lambda b, kv: (b, 0, 0)),
                pl.BlockSpec((_SEQ * _BPS, _BS), lambda b, kv: (b, 0)),
            ],
            out_specs=pl.BlockSpec((_SEQ, 1, KV), lambda b, kv: (b, 0, 0)),
        ),
        out_shape=jax.ShapeDtypeStruct((B, 1, KV), jnp.float32),
    )(kvl, q, k.reshape(B, 1, D), weights.reshape(B, 1, H),
      k_cache, k_s_cache)
    scores = scores.reshape(B, KV)

    vals, idx = pl.pallas_call(
        _topk_body,
        grid=(B // _R,),
        in_specs=[
            pl.BlockSpec((_R, 1), lambda i: (i, 0)),
            pl.BlockSpec((_R, KV), lambda i: (i, 0)),
        ],
        out_specs=[
            pl.BlockSpec((_R, _TOPK), lambda i: (i, 0)),
            pl.BlockSpec((_R, _TOPK), lambda i: (i, 0)),
        ],
        out_shape=[
            jax.ShapeDtypeStruct((B, _TOPK), jnp.float32),
            jax.ShapeDtypeStruct((B, _TOPK), jnp.int32),
        ],
    )(kvl.reshape(B, 1), scores)

    return idx, vals


# speedup vs baseline: 1.4346x; 1.0130x over previous
import functools

import jax
import jax.numpy as jnp
from jax.experimental import pallas as pl
from jax.experimental.pallas import tpu as pltpu

_TOPK = 2048
_SCALE = 0.08838834764831845
_NEG = -1e30
_FILL = -1

_B, _H, _D, _KV = 64, 32, 128, 4096
_BS = 128
_BPS = _KV // _BS


_SEQ = 2


def _scores_body(kvlen_ref, qT_ref, k_ref, w_ref, kc_ref, ks_ref, out_ref):
    b = pl.program_id(0)
    col = jax.lax.broadcasted_iota(jnp.int32, (1, _KV), 1)
    for s in range(_SEQ):
        seqlen = jnp.maximum(kvlen_ref[b * _SEQ + s], 1)
        p = seqlen - 1
        kd = (kc_ref[s * _BPS:(s + 1) * _BPS]
              * ks_ref[s * _BPS:(s + 1) * _BPS][:, :, None])
        K2 = kd.reshape(_KV, _D)
        qh = qT_ref[s]
        ltT = jax.lax.dot_general(K2, qh, (((1,), (1,)), ((), ())),
                                  preferred_element_type=jnp.float32)
        r = jnp.maximum(ltT * _SCALE, 0.0).astype(jnp.bfloat16)
        w = w_ref[s].astype(jnp.bfloat16)
        scores = jax.lax.dot_general(w, r, (((1,), (1,)), ((), ())),
                                     preferred_element_type=jnp.float32)
        kb = k_ref[s]
        qk = jax.lax.dot_general(kb, qh, (((1,), (1,)), ((), ())),
                                 preferred_element_type=jnp.float32)
        rc = jnp.maximum(qk * _SCALE, 0.0).astype(jnp.bfloat16)
        cur = jax.lax.dot_general(rc, w, (((1,), (1,)), ((), ())),
                                  preferred_element_type=jnp.float32)
        scores = jnp.where(col == p, cur, scores)
        out_ref[s] = jnp.where(col < seqlen, scores, _NEG)


_R = 16
_HALF = 2048


def _cmpx(keys, idx, j, is_lower, keep_max):
    pk = jnp.where(is_lower, jnp.roll(keys, -j, axis=1),
                   jnp.roll(keys, j, axis=1))
    pi = jnp.where(is_lower, jnp.roll(idx, -j, axis=1),
                   jnp.roll(idx, j, axis=1))
    beats = (pk > keys) | ((pk == keys) & (pi < idx))
    sel = ~(beats ^ keep_max)
    return jnp.where(sel, pk, keys), jnp.where(sel, pi, idx)


def _topk_body(kv_ref, s_ref, vals_ref, idx_ref):
    s = s_ref[...]
    colv = jax.lax.broadcasted_iota(jnp.int32, (_R, _HALF), 1)
    ka = s[:, :_HALF]
    kb = s[:, _HALF:]
    ia = colv
    ib = colv + _HALF
    k = 2
    while k <= _HALF:
        j = k // 2
        while j >= 1:
            is_lower = (colv & j) == 0
            km = ((colv & k) == 0) == is_lower
            ka, ia = _cmpx(ka, ia, j, is_lower, km)
            kb, ib = _cmpx(kb, ib, j, is_lower, ~km)
            j //= 2
        k *= 2
    bwins = (kb > ka) | ((kb == ka) & (ib < ia))
    keys = jnp.where(bwins, kb, ka)
    idx = jnp.where(bwins, ib, ia)
    j = _HALF // 2
    while j >= 1:
        is_lower = (colv & j) == 0
        keys, idx = _cmpx(keys, idx, j, is_lower, is_lower)
        j //= 2
    kv = jnp.maximum(kv_ref[...], 1)
    ranks = jax.lax.broadcasted_iota(jnp.int32, (_R, _TOPK), 1)
    vals_ref[...] = keys
    idx_ref[...] = jnp.where(ranks < jnp.minimum(kv, _TOPK), idx, _FILL)


def kernel(q, k, weights, k_cache, k_s_cache, block_offsets, kv_seqlens):
    B, H, D = q.shape
    KV = block_offsets.shape[1] * k_cache.shape[1]
    kvl = kv_seqlens.astype(jnp.int32)

    scores = pl.pallas_call(
        _scores_body,
        grid_spec=pltpu.PrefetchScalarGridSpec(
            num_scalar_prefetch=1,
            grid=(B // _SEQ,),
            in_specs=[
                pl.BlockSpec((_SEQ, H, D), lambda b, kv: (b, 0, 0)),
                pl.BlockSpec((_SEQ, 1, D), lambda b, kv: (b, 0, 0)),
                pl.BlockSpec((_SEQ, 1, H), lambda b, kv: (b, 0, 0)),
                pl.BlockSpec((_SEQ * _BPS, _BS, D), lambda b, kv: (b, 0, 0)),
                pl.BlockSpec((_SEQ * _BPS, _BS), lambda b, kv: (b, 0)),
            ],
            out_specs=pl.BlockSpec((_SEQ, 1, KV), lambda b, kv: (b, 0, 0)),
        ),
        out_shape=jax.ShapeDtypeStruct((B, 1, KV), jnp.float32),
    )(kvl, q, k.reshape(B, 1, D), weights.reshape(B, 1, H),
      k_cache, k_s_cache)
    scores = scores.reshape(B, KV)

    vals, idx = pl.pallas_call(
        _topk_body,
        grid=(B // _R,),
        in_specs=[
            pl.BlockSpec((_R, 1), lambda i: (i, 0)),
            pl.BlockSpec((_R, KV), lambda i: (i, 0)),
        ],
        out_specs=[
            pl.BlockSpec((_R, _TOPK), lambda i: (i, 0)),
            pl.BlockSpec((_R, _TOPK), lambda i: (i, 0)),
        ],
        out_shape=[
            jax.ShapeDtypeStruct((B, _TOPK), jnp.float32),
            jax.ShapeDtypeStruct((B, _TOPK), jnp.int32),
        ],
    )(kvl.reshape(B, 1), scores)

    return idx, vals


# speedup vs baseline: 1.5725x; 1.0961x over previous
import functools

import jax
import jax.numpy as jnp
from jax.experimental import pallas as pl
from jax.experimental.pallas import tpu as pltpu

_TOPK = 2048
_SCALE = 0.08838834764831845
_NEG = -1e30
_FILL = -1

_B, _H, _D, _KV = 64, 32, 128, 4096
_BS = 128
_BPS = _KV // _BS

_G = 8
_HALF = 2048


def _cmpx(keys, idx, j, is_lower, keep_max):
    pk = jnp.where(is_lower, jnp.roll(keys, -j, axis=1),
                   jnp.roll(keys, j, axis=1))
    pi = jnp.where(is_lower, jnp.roll(idx, -j, axis=1),
                   jnp.roll(idx, j, axis=1))
    beats = (pk > keys) | ((pk == keys) & (pi < idx))
    sel = ~(beats ^ keep_max)
    return jnp.where(sel, pk, keys), jnp.where(sel, pi, idx)


def _body(kvlen_ref, qT_ref, k_ref, w_ref, kc_ref, ks_ref,
          vals_ref, idx_ref):
    b = pl.program_id(0)
    col = jax.lax.broadcasted_iota(jnp.int32, (1, _KV), 1)
    rows = []
    kvs = []
    for s in range(_G):
        seqlen = jnp.maximum(kvlen_ref[b * _G + s], 1)
        kvs.append(seqlen)
        p = seqlen - 1
        kd = (kc_ref[s * _BPS:(s + 1) * _BPS]
              * ks_ref[s * _BPS:(s + 1) * _BPS][:, :, None])
        K2 = kd.reshape(_KV, _D)
        qh = qT_ref[s]
        ltT = jax.lax.dot_general(K2, qh, (((1,), (1,)), ((), ())),
                                  preferred_element_type=jnp.float32)
        r = jnp.maximum(ltT * _SCALE, 0.0).astype(jnp.bfloat16)
        w = w_ref[s].astype(jnp.bfloat16)
        scores = jax.lax.dot_general(w, r, (((1,), (1,)), ((), ())),
                                     preferred_element_type=jnp.float32)
        kb = k_ref[s]
        qk = jax.lax.dot_general(kb, qh, (((1,), (1,)), ((), ())),
                                 preferred_element_type=jnp.float32)
        rc = jnp.maximum(qk * _SCALE, 0.0).astype(jnp.bfloat16)
        cur = jax.lax.dot_general(rc, w, (((1,), (1,)), ((), ())),
                                  preferred_element_type=jnp.float32)
        scores = jnp.where(col == p, cur, scores)
        rows.append(jnp.where(col < seqlen, scores, _NEG))
    s_all = jnp.concatenate(rows, axis=0)

    colv = jax.lax.broadcasted_iota(jnp.int32, (_G, _HALF), 1)
    ka = s_all[:, :_HALF]
    kb2 = s_all[:, _HALF:]
    ia = colv
    ib = colv + _HALF
    k = 2
    while k <= _HALF:
        j = k // 2
        while j >= 1:
            is_lower = (colv & j) == 0
            km = ((colv & k) == 0) == is_lower
            ka, ia = _cmpx(ka, ia, j, is_lower, km)
            kb2, ib = _cmpx(kb2, ib, j, is_lower, ~km)
            j //= 2
        k *= 2
    bwins = (kb2 > ka) | ((kb2 == ka) & (ib < ia))
    keys = jnp.where(bwins, kb2, ka)
    idx = jnp.where(bwins, ib, ia)
    j = _HALF // 2
    while j >= 1:
        is_lower = (colv & j) == 0
        keys, idx = _cmpx(keys, idx, j, is_lower, is_lower)
        j //= 2
    kv = jnp.stack(kvs).reshape(_G, 1)
    ranks = jax.lax.broadcasted_iota(jnp.int32, (_G, _TOPK), 1)
    vals_ref[...] = keys
    idx_ref[...] = jnp.where(ranks < jnp.minimum(kv, _TOPK), idx, _FILL)


def kernel(q, k, weights, k_cache, k_s_cache, block_offsets, kv_seqlens):
    B, H, D = q.shape
    KV = block_offsets.shape[1] * k_cache.shape[1]
    kvl = kv_seqlens.astype(jnp.int32)

    vals, idx = pl.pallas_call(
        _body,
        grid_spec=pltpu.PrefetchScalarGridSpec(
            num_scalar_prefetch=1,
            grid=(B // _G,),
            in_specs=[
                pl.BlockSpec((_G, H, D), lambda b, kv: (b, 0, 0)),
                pl.BlockSpec((_G, 1, D), lambda b, kv: (b, 0, 0)),
                pl.BlockSpec((_G, 1, H), lambda b, kv: (b, 0, 0)),
                pl.BlockSpec((_G * _BPS, _BS, D), lambda b, kv: (b, 0, 0)),
                pl.BlockSpec((_G * _BPS, _BS), lambda b, kv: (b, 0)),
            ],
            out_specs=[
                pl.BlockSpec((_G, _TOPK), lambda b, kv: (b, 0)),
                pl.BlockSpec((_G, _TOPK), lambda b, kv: (b, 0)),
            ],
        ),
        out_shape=[
            jax.ShapeDtypeStruct((B, _TOPK), jnp.float32),
            jax.ShapeDtypeStruct((B, _TOPK), jnp.int32),
        ],
    )(kvl, q, k.reshape(B, 1, D), weights.reshape(B, 1, H),
      k_cache, k_s_cache)

    return idx, vals
